# baseline (speedup 1.0000x reference)
import functools

import jax
import jax.numpy as jnp
from jax import lax
from jax.experimental import pallas as pl
from jax.experimental.pallas import tpu as pltpu
from jax.experimental.pallas import tpu_sc as plsc

_B, _N, _D, _R = 4, 2048, 256, 3
_RN = _R * _N
_BN = 256
_NB = _N // _BN
_KEEP_RATIO = 0.5


def _fused_body(bias_ref, w2_ref, adj_ref, k_ref, nodes_ref, nodesf_ref,
                s_ref, ranks_ref, hidden_ref,
                srow_scr, scol_scr, vcol_scr):
    b = pl.program_id(0)
    nb = pl.program_id(1)

    @pl.when((b < _B) & (nb == 0))
    def _xw_stage():
        xwb = lax.dot_general(nodesf_ref[0], w2_ref[...],
                              (((1,), (0,)), ((), ())),
                              preferred_element_type=jnp.float32)
        xw_bf = xwb.astype(jnp.bfloat16)
        vcol_scr[...] = jnp.concatenate(
            [xw_bf[:, 0:1], xw_bf[:, 1:2], xw_bf[:, 2:3]], axis=0)

    @pl.when(b > 0)
    def _rank_stage():
        srow = srow_scr[(b - 1) % 2]
        scol = scol_scr[pl.ds(nb * _BN, _BN), :]
        jpos = lax.broadcasted_iota(jnp.int32, (_BN, _N), 1)
        ipos = lax.broadcasted_iota(jnp.int32, (_BN, _N), 0) + nb * _BN
        before = (srow > scol) | ((srow == scol) & (jpos < ipos))
        ones_n = jnp.ones((_N, 1), jnp.float32)
        cnt = lax.dot_general(before.astype(jnp.float32), ones_n,
                              (((1,), (0,)), ((), ())),
                              preferred_element_type=jnp.float32)
        ranks_col = cnt.astype(jnp.int32)
        mask_col = ranks_col < k_ref[0, 0, 0]
        ranksm = jnp.where(mask_col, ranks_col, _N)
        ranks_ref[pl.ds(nb * _BN, _BN)] = ranksm.reshape(_BN)
        sm_col = jnp.where(mask_col, scol, 0.0)
        hidden_ref[0] = nodes_ref[0] * sm_col

    @pl.when(b < _B)
    def _conv_stage():
        a = adj_ref[0]
        acat = jnp.concatenate([a[0], a[1], a[2]], axis=1)
        vcol = vcol_scr[...]
        x = lax.dot_general(acat, vcol, (((1,), (0,)), ((), ())),
                            preferred_element_type=jnp.float32)
        s_col = jnp.tanh(x + bias_ref[0, 0])
        s_row = s_col.reshape(_BN)
        s_ref[pl.ds(nb * _BN, _BN)] = s_row
        srow_scr[b % 2, 0, pl.ds(nb * _BN, _BN)] = s_row
        scol_scr[pl.ds(nb * _BN, _BN), :] = s_col


def _fused(bias, w2_bf, adjacency, keep3, nodes):
    return pl.pallas_call(
        _fused_body,
        grid=(_B + 1, _NB),
        in_specs=[
            pl.BlockSpec((1, 1), lambda b, nb: (0, 0)),
            pl.BlockSpec((_D, _R), lambda b, nb: (0, 0)),
            pl.BlockSpec((1, _R, _BN, _N),
                         lambda b, nb: (jnp.minimum(b, _B - 1), 0,
                                        jnp.where(b == _B, _NB - 1, nb), 0)),
            pl.BlockSpec((1, 1, 1), lambda b, nb: (jnp.maximum(b - 1, 0), 0, 0)),
            pl.BlockSpec((1, _BN, _D),
                         lambda b, nb: (jnp.maximum(b - 1, 0),
                                        jnp.where(b == 0, 0, nb), 0)),
            pl.BlockSpec((1, _N, _D), lambda b, nb: (jnp.minimum(b, _B - 1), 0, 0)),
        ],
        out_specs=[
            pl.BlockSpec((_N,), lambda b, nb: (jnp.minimum(b, _B - 1),)),
            pl.BlockSpec((_N,), lambda b, nb: (jnp.maximum(b - 1, 0),)),
            pl.BlockSpec((1, _BN, _D),
                         lambda b, nb: (jnp.maximum(b - 1, 0),
                                        jnp.where(b == 0, 0, nb), 0)),
        ],
        out_shape=[
            jax.ShapeDtypeStruct((_B * _N,), jnp.float32),
            jax.ShapeDtypeStruct((_B * _N,), jnp.int32),
            jax.ShapeDtypeStruct((_B, _N, _D), jnp.float32),
        ],
        scratch_shapes=[
            pltpu.VMEM((2, 1, _N), jnp.float32),
            pltpu.VMEM((_N, 1), jnp.float32),
            pltpu.VMEM((_RN, 1), jnp.bfloat16),
        ],
    )(bias, w2_bf, adjacency, keep3, nodes, nodes)


def _sc_scatter(ranks_f, scores_f):
    info = plsc.get_sparse_core_info()
    nw = info.num_cores * info.num_subcores
    lanes = info.num_lanes
    chunks = nw // _B
    cw = _N // chunks

    @functools.partial(
        pl.kernel,
        out_type=(jax.ShapeDtypeStruct((_B, _N), jnp.int32),
                  jax.ShapeDtypeStruct((_B, _N), jnp.float32)),
        mesh=plsc.VectorSubcoreMesh(core_axis_name="c", subcore_axis_name="s"),
        compiler_params=pltpu.CompilerParams(needs_layout_passes=False),
        scratch_types=[
            pltpu.VMEM((_N,), jnp.int32),
            pltpu.VMEM((_N,), jnp.float32),
            pltpu.VMEM((_N,), jnp.int32),
            pltpu.VMEM((_N,), jnp.float32),
            pltpu.SemaphoreType.DMA,
        ],
    )
    def scatter_kernel(ranks_hbm, scores_hbm, kni_hbm, kns_hbm,
                       rk_v, sc_v, oi_v, os_v, sem):
        wid = lax.axis_index("s") * info.num_cores + lax.axis_index("c")
        bb = wid // chunks
        ck = wid % chunks
        c1 = pltpu.async_copy(ranks_hbm.at[pl.ds(bb * _N, _N)], rk_v, sem)
        c2 = pltpu.async_copy(scores_hbm.at[pl.ds(bb * _N, _N)], sc_v, sem)

        def init_body(i, carry):
            oi_v[pl.ds(i * lanes, lanes)] = jnp.full((lanes,), -1, jnp.int32)
            os_v[pl.ds(i * lanes, lanes)] = jnp.zeros((lanes,), jnp.float32)
            return carry

        lax.fori_loop(0, _N // lanes, init_body, 0)
        c1.wait()
        c2.wait()
        base_iota = lax.iota(jnp.int32, lanes)

        def scat_body(i, carry):
            idx = rk_v[pl.ds(i * lanes, lanes)]
            m = idx < _N
            plsc.store_scatter(oi_v, [idx], base_iota + i * lanes, mask=m)
            plsc.store_scatter(os_v, [idx], sc_v[pl.ds(i * lanes, lanes)], mask=m)
            return carry

        lax.fori_loop(0, _N // lanes, scat_body, 0)
        pltpu.sync_copy(oi_v.at[pl.ds(ck * cw, cw)],
                        kni_hbm.at[bb, pl.ds(ck * cw, cw)])
        pltpu.sync_copy(os_v.at[pl.ds(ck * cw, cw)],
                        kns_hbm.at[bb, pl.ds(ck * cw, cw)])

    return scatter_kernel(ranks_f, scores_f)


def kernel(nodes, adjacency, batch_node_nums, W, b):
    w2_bf = W.reshape(_R, _D).T.astype(jnp.bfloat16)
    bias = b.reshape(1, 1)

    keep_num = jnp.ceil(_KEEP_RATIO * batch_node_nums.astype(jnp.float32)).astype(jnp.int32)
    keep3 = keep_num.reshape(_B, 1, 1)

    scores_f, ranks_f, hidden = _fused(bias, w2_bf, adjacency, keep3, nodes)
    kni, kns = _sc_scatter(ranks_f, scores_f)
    return (hidden, keep_num, kni, kns)


# speedup vs baseline: 1.0027x; 1.0027x over previous
import functools

import jax
import jax.numpy as jnp
from jax import lax
from jax.experimental import pallas as pl
from jax.experimental.pallas import tpu as pltpu
from jax.experimental.pallas import tpu_sc as plsc

_B, _N, _D, _R = 4, 2048, 256, 3
_RN = _R * _N
_BN = 256
_NB = _N // _BN
_KEEP_RATIO = 0.5


def _fused_body(bias_ref, w_ref, adj_ref, nums_ref, nodes_ref, nodesf_ref,
                s_ref, ranks_ref, hidden_ref,
                srow_scr, scol_scr, vcol_scr):
    b = pl.program_id(0)
    nb = pl.program_id(1)

    @pl.when((b < _B) & (nb == 0))
    def _xw_stage():
        w2 = jnp.transpose(w_ref[..., 0], (1, 0)).astype(jnp.bfloat16)
        xwb = lax.dot_general(nodesf_ref[0], w2,
                              (((1,), (0,)), ((), ())),
                              preferred_element_type=jnp.float32)
        xw_bf = xwb.astype(jnp.bfloat16)
        vcol_scr[...] = jnp.concatenate(
            [xw_bf[:, 0:1], xw_bf[:, 1:2], xw_bf[:, 2:3]], axis=0)

    @pl.when(b > 0)
    def _rank_stage():
        srow = srow_scr[(b - 1) % 2]
        scol = scol_scr[pl.ds(nb * _BN, _BN), :]
        jpos = lax.broadcasted_iota(jnp.int32, (_BN, _N), 1)
        ipos = lax.broadcasted_iota(jnp.int32, (_BN, _N), 0) + nb * _BN
        before = (srow > scol) | ((srow == scol) & (jpos < ipos))
        ones_n = jnp.ones((_N, 1), jnp.float32)
        cnt = lax.dot_general(before.astype(jnp.float32), ones_n,
                              (((1,), (0,)), ((), ())),
                              preferred_element_type=jnp.float32)
        ranks_col = cnt.astype(jnp.int32)
        k = (nums_ref[b - 1] + 1) // 2
        mask_col = ranks_col < k
        ranksm = jnp.where(mask_col, ranks_col, _N)
        ranks_ref[pl.ds(nb * _BN, _BN)] = ranksm.reshape(_BN)
        sm_col = jnp.where(mask_col, scol, 0.0)
        hidden_ref[0] = nodes_ref[0] * sm_col

    @pl.when(b < _B)
    def _conv_stage():
        a = adj_ref[0]
        acat = jnp.concatenate([a[0], a[1], a[2]], axis=1)
        vcol = vcol_scr[...]
        x = lax.dot_general(acat, vcol, (((1,), (0,)), ((), ())),
                            preferred_element_type=jnp.float32)
        s_col = jnp.tanh(x + bias_ref[0])
        s_row = s_col.reshape(_BN)
        s_ref[pl.ds(nb * _BN, _BN)] = s_row
        srow_scr[b % 2, 0, pl.ds(nb * _BN, _BN)] = s_row
        scol_scr[pl.ds(nb * _BN, _BN), :] = s_col


def _fused(bias, W, adjacency, nums, nodes):
    return pl.pallas_call(
        _fused_body,
        grid=(_B + 1, _NB),
        in_specs=[
            pl.BlockSpec(memory_space=pltpu.SMEM),
            pl.BlockSpec((_R, _D, 1), lambda b, nb: (0, 0, 0)),
            pl.BlockSpec((1, _R, _BN, _N),
                         lambda b, nb: (jnp.minimum(b, _B - 1), 0,
                                        jnp.where(b == _B, _NB - 1, nb), 0)),
            pl.BlockSpec(memory_space=pltpu.SMEM),
            pl.BlockSpec((1, _BN, _D),
                         lambda b, nb: (jnp.maximum(b - 1, 0),
                                        jnp.where(b == 0, 0, nb), 0)),
            pl.BlockSpec((1, _N, _D), lambda b, nb: (jnp.minimum(b, _B - 1), 0, 0)),
        ],
        out_specs=[
            pl.BlockSpec((_N,), lambda b, nb: (jnp.minimum(b, _B - 1),)),
            pl.BlockSpec((_N,), lambda b, nb: (jnp.maximum(b - 1, 0),)),
            pl.BlockSpec((1, _BN, _D),
                         lambda b, nb: (jnp.maximum(b - 1, 0),
                                        jnp.where(b == 0, 0, nb), 0)),
        ],
        out_shape=[
            jax.ShapeDtypeStruct((_B * _N,), jnp.float32),
            jax.ShapeDtypeStruct((_B * _N,), jnp.int32),
            jax.ShapeDtypeStruct((_B, _N, _D), jnp.float32),
        ],
        scratch_shapes=[
            pltpu.VMEM((2, 1, _N), jnp.float32),
            pltpu.VMEM((_N, 1), jnp.float32),
            pltpu.VMEM((_RN, 1), jnp.bfloat16),
        ],
    )(bias, W, adjacency, nums, nodes, nodes)


def _sc_scatter(ranks_f, scores_f):
    info = plsc.get_sparse_core_info()
    nw = info.num_cores * info.num_subcores
    lanes = info.num_lanes
    chunks = nw // _B
    cw = _N // chunks

    @functools.partial(
        pl.kernel,
        out_type=(jax.ShapeDtypeStruct((_B, _N), jnp.int32),
                  jax.ShapeDtypeStruct((_B, _N), jnp.float32)),
        mesh=plsc.VectorSubcoreMesh(core_axis_name="c", subcore_axis_name="s"),
        compiler_params=pltpu.CompilerParams(needs_layout_passes=False),
        scratch_types=[
            pltpu.VMEM((_N,), jnp.int32),
            pltpu.VMEM((_N,), jnp.float32),
            pltpu.VMEM((_N,), jnp.int32),
            pltpu.VMEM((_N,), jnp.float32),
            pltpu.SemaphoreType.DMA,
        ],
    )
    def scatter_kernel(ranks_hbm, scores_hbm, kni_hbm, kns_hbm,
                       rk_v, sc_v, oi_v, os_v, sem):
        wid = lax.axis_index("s") * info.num_cores + lax.axis_index("c")
        bb = wid // chunks
        ck = wid % chunks
        c1 = pltpu.async_copy(ranks_hbm.at[pl.ds(bb * _N, _N)], rk_v, sem)
        c2 = pltpu.async_copy(scores_hbm.at[pl.ds(bb * _N, _N)], sc_v, sem)

        def init_body(i, carry):
            oi_v[pl.ds(i * lanes, lanes)] = jnp.full((lanes,), -1, jnp.int32)
            os_v[pl.ds(i * lanes, lanes)] = jnp.zeros((lanes,), jnp.float32)
            return carry

        lax.fori_loop(0, _N // lanes, init_body, 0)
        c1.wait()
        c2.wait()
        base_iota = lax.iota(jnp.int32, lanes)
        unroll = 4

        def scat_body(i, carry):
            for u in range(unroll):
                off = i * (lanes * unroll) + u * lanes
                idx = rk_v[pl.ds(off, lanes)]
                m = idx < _N
                plsc.store_scatter(oi_v, [idx], base_iota + off, mask=m)
                plsc.store_scatter(os_v, [idx], sc_v[pl.ds(off, lanes)], mask=m)
            return carry

        lax.fori_loop(0, _N // (lanes * unroll), scat_body, 0)
        o1 = pltpu.async_copy(oi_v.at[pl.ds(ck * cw, cw)],
                              kni_hbm.at[bb, pl.ds(ck * cw, cw)], sem)
        o2 = pltpu.async_copy(os_v.at[pl.ds(ck * cw, cw)],
                              kns_hbm.at[bb, pl.ds(ck * cw, cw)], sem)
        o1.wait()
        o2.wait()

    return scatter_kernel(ranks_f, scores_f)


def kernel(nodes, adjacency, batch_node_nums, W, b):
    keep_num = jnp.ceil(_KEEP_RATIO * batch_node_nums.astype(jnp.float32)).astype(jnp.int32)

    scores_f, ranks_f, hidden = _fused(b, W, adjacency, batch_node_nums, nodes)
    kni, kns = _sc_scatter(ranks_f, scores_f)
    return (hidden, keep_num, kni, kns)


# speedup vs baseline: 1.1189x; 1.1159x over previous
import functools

import jax
import jax.numpy as jnp
from jax import lax
from jax.experimental import pallas as pl
from jax.experimental.pallas import tpu as pltpu
from jax.experimental.pallas import tpu_sc as plsc

_B, _N, _D, _R = 4, 2048, 256, 3
_RN = _R * _N
_BN = 512
_NB = _N // _BN
_KEEP_RATIO = 0.5


def _fused_body(bias_ref, w_ref, adj_ref, nums_ref, nodes_ref, nodesf_ref,
                s_ref, ranks_ref, hidden_ref,
                srow_scr, scol_scr, vcol_scr):
    b = pl.program_id(0)
    nb = pl.program_id(1)

    @pl.when((b < _B) & (nb == 0))
    def _xw_stage():
        w2 = jnp.transpose(w_ref[..., 0], (1, 0)).astype(jnp.bfloat16)
        xwb = lax.dot_general(nodesf_ref[0], w2,
                              (((1,), (0,)), ((), ())),
                              preferred_element_type=jnp.float32)
        xw_bf = xwb.astype(jnp.bfloat16)
        vcol_scr[...] = jnp.concatenate(
            [xw_bf[:, 0:1], xw_bf[:, 1:2], xw_bf[:, 2:3]], axis=0)

    @pl.when(b > 0)
    def _rank_stage():
        srow = srow_scr[(b - 1) % 2]
        scol = scol_scr[pl.ds(nb * _BN, _BN), :]
        jpos = lax.broadcasted_iota(jnp.int32, (_BN, _N), 1)
        ipos = lax.broadcasted_iota(jnp.int32, (_BN, _N), 0) + nb * _BN
        before = (srow > scol) | ((srow == scol) & (jpos < ipos))
        ones_n = jnp.ones((_N, 1), jnp.float32)
        cnt = lax.dot_general(before.astype(jnp.float32), ones_n,
                              (((1,), (0,)), ((), ())),
                              preferred_element_type=jnp.float32)
        ranks_col = cnt.astype(jnp.int32)
        k = (nums_ref[b - 1] + 1) // 2
        mask_col = ranks_col < k
        ranksm = jnp.where(mask_col, ranks_col, _N)
        ranks_ref[pl.ds(nb * _BN, _BN)] = ranksm.reshape(_BN)
        sm_col = jnp.where(mask_col, scol, 0.0)
        hidden_ref[0] = nodes_ref[0] * sm_col

    @pl.when(b < _B)
    def _conv_stage():
        a = adj_ref[0]
        acat = jnp.concatenate([a[0], a[1], a[2]], axis=1)
        vcol = vcol_scr[...]
        x = lax.dot_general(acat, vcol, (((1,), (0,)), ((), ())),
                            preferred_element_type=jnp.float32)
        s_col = jnp.tanh(x + bias_ref[0])
        s_row = s_col.reshape(_BN)
        s_ref[pl.ds(nb * _BN, _BN)] = s_row
        srow_scr[b % 2, 0, pl.ds(nb * _BN, _BN)] = s_row
        scol_scr[pl.ds(nb * _BN, _BN), :] = s_col


def _fused(bias, W, adjacency, nums, nodes):
    return pl.pallas_call(
        _fused_body,
        grid=(_B + 1, _NB),
        in_specs=[
            pl.BlockSpec(memory_space=pltpu.SMEM),
            pl.BlockSpec((_R, _D, 1), lambda b, nb: (0, 0, 0)),
            pl.BlockSpec((1, _R, _BN, _N),
                         lambda b, nb: (jnp.minimum(b, _B - 1), 0,
                                        jnp.where(b == _B, _NB - 1, nb), 0)),
            pl.BlockSpec(memory_space=pltpu.SMEM),
            pl.BlockSpec((1, _BN, _D),
                         lambda b, nb: (jnp.maximum(b - 1, 0),
                                        jnp.where(b == 0, 0, nb), 0)),
            pl.BlockSpec((1, _N, _D), lambda b, nb: (jnp.minimum(b, _B - 1), 0, 0)),
        ],
        out_specs=[
            pl.BlockSpec((_N,), lambda b, nb: (jnp.minimum(b, _B - 1),)),
            pl.BlockSpec((_N,), lambda b, nb: (jnp.maximum(b - 1, 0),)),
            pl.BlockSpec((1, _BN, _D),
                         lambda b, nb: (jnp.maximum(b - 1, 0),
                                        jnp.where(b == 0, 0, nb), 0)),
        ],
        out_shape=[
            jax.ShapeDtypeStruct((_B * _N,), jnp.float32),
            jax.ShapeDtypeStruct((_B * _N,), jnp.int32),
            jax.ShapeDtypeStruct((_B, _N, _D), jnp.float32),
        ],
        scratch_shapes=[
            pltpu.VMEM((2, 1, _N), jnp.float32),
            pltpu.VMEM((_N, 1), jnp.float32),
            pltpu.VMEM((_RN, 1), jnp.bfloat16),
        ],
    )(bias, W, adjacency, nums, nodes, nodes)


def _sc_scatter(ranks_f, scores_f):
    info = plsc.get_sparse_core_info()
    nw = info.num_cores * info.num_subcores
    lanes = info.num_lanes
    chunks = nw // _B
    cw = _N // chunks

    @functools.partial(
        pl.kernel,
        out_type=(jax.ShapeDtypeStruct((_B, _N), jnp.int32),
                  jax.ShapeDtypeStruct((_B, _N), jnp.float32)),
        mesh=plsc.VectorSubcoreMesh(core_axis_name="c", subcore_axis_name="s"),
        compiler_params=pltpu.CompilerParams(needs_layout_passes=False),
        scratch_types=[
            pltpu.VMEM((_N,), jnp.int32),
            pltpu.VMEM((_N,), jnp.float32),
            pltpu.VMEM((_N,), jnp.int32),
            pltpu.VMEM((_N,), jnp.float32),
            pltpu.SemaphoreType.DMA,
        ],
    )
    def scatter_kernel(ranks_hbm, scores_hbm, kni_hbm, kns_hbm,
                       rk_v, sc_v, oi_v, os_v, sem):
        wid = lax.axis_index("s") * info.num_cores + lax.axis_index("c")
        bb = wid // chunks
        ck = wid % chunks
        c1 = pltpu.async_copy(ranks_hbm.at[pl.ds(bb * _N, _N)], rk_v, sem)
        c2 = pltpu.async_copy(scores_hbm.at[pl.ds(bb * _N, _N)], sc_v, sem)

        def init_body(i, carry):
            oi_v[pl.ds(i * lanes, lanes)] = jnp.full((lanes,), -1, jnp.int32)
            os_v[pl.ds(i * lanes, lanes)] = jnp.zeros((lanes,), jnp.float32)
            return carry

        lax.fori_loop(0, _N // lanes, init_body, 0)
        c1.wait()
        c2.wait()
        base_iota = lax.iota(jnp.int32, lanes)
        unroll = 4

        def scat_body(i, carry):
            for u in range(unroll):
                off = i * (lanes * unroll) + u * lanes
                idx = rk_v[pl.ds(off, lanes)]
                m = idx < _N
                plsc.store_scatter(oi_v, [idx], base_iota + off, mask=m)
                plsc.store_scatter(os_v, [idx], sc_v[pl.ds(off, lanes)], mask=m)
            return carry

        lax.fori_loop(0, _N // (lanes * unroll), scat_body, 0)
        o1 = pltpu.async_copy(oi_v.at[pl.ds(ck * cw, cw)],
                              kni_hbm.at[bb, pl.ds(ck * cw, cw)], sem)
        o2 = pltpu.async_copy(os_v.at[pl.ds(ck * cw, cw)],
                              kns_hbm.at[bb, pl.ds(ck * cw, cw)], sem)
        o1.wait()
        o2.wait()

    return scatter_kernel(ranks_f, scores_f)


def kernel(nodes, adjacency, batch_node_nums, W, b):
    keep_num = jnp.ceil(_KEEP_RATIO * batch_node_nums.astype(jnp.float32)).astype(jnp.int32)

    scores_f, ranks_f, hidden = _fused(b, W, adjacency, batch_node_nums, nodes)
    kni, kns = _sc_scatter(ranks_f, scores_f)
    return (hidden, keep_num, kni, kns)
